# speedup vs baseline: 1.8237x; 1.8237x over previous
import functools

import jax
import jax.numpy as jnp
from jax import lax
from jax.experimental import pallas as pl
from jax.experimental.pallas import tpu as pltpu
from jax.experimental.pallas import tpu_sc as plsc

NC = 2
NS = 16
NW = NC * NS

CHUNK = 16


def _make_gather(vocab, dim, n):
    assert n % NW == 0
    b_per_w = n // NW
    assert b_per_w % CHUNK == 0
    n_chunks = b_per_w // CHUNK

    mesh = plsc.VectorSubcoreMesh(core_axis_name="c", subcore_axis_name="s")

    @functools.partial(
        pl.kernel,
        out_type=jax.ShapeDtypeStruct((n, dim), jnp.float32),
        mesh=mesh,
        scratch_types=[
            pltpu.VMEM((b_per_w,), jnp.int32),
            pltpu.VMEM((CHUNK, dim), jnp.float32),
            pltpu.VMEM((CHUNK, dim), jnp.float32),
            pltpu.SemaphoreType.DMA,
            pltpu.SemaphoreType.DMA,
        ],
    )
    def gather(table_hbm, idx_hbm, out_hbm, idx_v, buf0, buf1, sem0, sem1):
        wid = lax.axis_index("s") * NC + lax.axis_index("c")
        base = wid * b_per_w
        pltpu.sync_copy(idx_hbm.at[pl.ds(base, b_per_w)], idx_v)

        bufs = (buf0, buf1)
        sems = (sem0, sem1)

        pltpu.async_copy(table_hbm.at[idx_v.at[pl.ds(0, CHUNK)]], buf0, sem0)

        def body(c, _):
            slot = lax.rem(c, 2)
            nxt = c + 1

            def start_next(s):
                @pl.when(nxt < n_chunks)
                def _():
                    pltpu.async_copy(
                        table_hbm.at[idx_v.at[pl.ds(nxt * CHUNK, CHUNK)]],
                        bufs[s],
                        sems[s],
                    )

            def drain_and_store(s):
                pltpu.make_async_copy(
                    table_hbm.at[pl.ds(0, CHUNK)], bufs[s], sems[s]
                ).wait()
                pltpu.sync_copy(bufs[s], out_hbm.at[pl.ds(base + c * CHUNK, CHUNK)])

            @pl.when(slot == 0)
            def _():
                start_next(1)
                drain_and_store(0)

            @pl.when(slot == 1)
            def _():
                start_next(0)
                drain_and_store(1)

            return 0

        lax.fori_loop(0, n_chunks, body, 0)

    return gather


def kernel(input_ids, weight):
    b, s = input_ids.shape
    vocab, dim = weight.shape
    idx = input_ids.reshape(-1).astype(jnp.int32)
    out = _make_gather(vocab, dim, idx.shape[0])(weight, idx)
    return out.reshape(b, s, dim)
